# baseline (speedup 1.0000x reference)
import functools

import jax
import jax.numpy as jnp
from jax import lax
from jax.experimental import pallas as pl
from jax.experimental.pallas import tpu as pltpu
from jax.experimental.pallas import tpu_sc as plsc

K_NN = 32
ROW_BLOCK = 256
DROP_BITS = 14
N_WORKERS = 32


def _mlp_body(pr_ref, w1_ref, b1_ref, w2_ref, b2_ref, w3_ref, b3_ref, f3_ref):
    p = pr_ref[0]
    f = jnp.maximum(jnp.dot(p, w1_ref[...], preferred_element_type=jnp.float32)
                    + b1_ref[...], 0.0)
    f = jnp.maximum(jnp.dot(f, w2_ref[...], preferred_element_type=jnp.float32)
                    + b2_ref[...], 0.0)
    f = jnp.maximum(jnp.dot(f, w3_ref[...], preferred_element_type=jnp.float32)
                    + b3_ref[...], 0.0)
    f3_ref[0] = f


def _lane_perm(v, idx):
    return lax.gather(
        v, idx[:, None],
        dimension_numbers=lax.GatherDimensionNumbers(
            offset_dims=(), collapsed_slice_dims=(0,), start_index_map=(0,)),
        slice_sizes=(1,),
        mode=lax.GatherScatterMode.PROMISE_IN_BOUNDS)


def _sc_kth_bits(x, y, z):
    B, N = x.shape
    rpw = (B * N) // N_WORKERS
    wpb = N // rpw
    n_chunks = N // 16
    mesh = plsc.VectorSubcoreMesh(core_axis_name="c", subcore_axis_name="s")

    @functools.partial(
        pl.kernel,
        out_type=(jax.ShapeDtypeStruct((N_WORKERS, rpw), jnp.int32),
                  jax.ShapeDtypeStruct((N_WORKERS, rpw), jnp.int32)),
        scratch_types=[
            pltpu.VMEM((N,), jnp.float32),
            pltpu.VMEM((N,), jnp.float32),
            pltpu.VMEM((N,), jnp.float32),
            pltpu.VMEM((16 * N,), jnp.float32),
            pltpu.VMEM((rpw,), jnp.int32),
            pltpu.VMEM((rpw,), jnp.int32),
        ],
        mesh=mesh,
    )
    def k(x_hbm, y_hbm, z_hbm, lo_hbm, hi_hbm, xv, yv, zv, d2v, lov, hiv):
        wid = lax.axis_index("s") * 2 + lax.axis_index("c")
        b = wid // wpb
        r0 = (wid % wpb) * rpw
        pltpu.sync_copy(x_hbm.at[b], xv)
        pltpu.sync_copy(y_hbm.at[b], yv)
        pltpu.sync_copy(z_hbm.at[b], zv)

        lane_ids = lax.iota(jnp.int32, 16)

        def group_body(g, carry):
            q0 = r0 + g * 16
            qx = xv[pl.ds(q0, 16)]
            qy = yv[pl.ds(q0, 16)]
            qz = zv[pl.ds(q0, 16)]

            def chunk(c, carry2):
                o = c * 16
                px = xv[pl.ds(o, 16)]
                py = yv[pl.ds(o, 16)]
                pz = zv[pl.ds(o, 16)]

                def rot(s, carry3):
                    idx = (lane_ids + s) & 15
                    dx = _lane_perm(px, idx) - qx
                    dy = _lane_perm(py, idx) - qy
                    dz = _lane_perm(pz, idx) - qz
                    d2v[pl.ds((c * 16 + s) * 16, 16)] = \
                        dx * dx + dy * dy + dz * dz
                    return carry3

                lax.fori_loop(0, 16, rot, 0, unroll=4)
                return carry2

            lax.fori_loop(0, n_chunks, chunk, 0)

            def sbody(i, t):
                shift = jnp.left_shift(jnp.int32(1), 30 - i)
                cand = t | jnp.full((16,), shift, jnp.int32)

                def cbody(c, cacc):
                    bits = lax.bitcast_convert_type(
                        d2v[pl.ds(c * 16, 16)], jnp.int32)
                    return cacc + jnp.where(bits < cand, 1, 0).astype(
                        jnp.int32)

                cntv = lax.fori_loop(0, N, cbody,
                                     jnp.zeros((16,), jnp.int32), unroll=8)
                return jnp.where(cntv < K_NN, cand, t)

            t = lax.fori_loop(0, 31 - DROP_BITS, sbody,
                              jnp.zeros((16,), jnp.int32))
            lov[pl.ds(g * 16, 16)] = t
            hiv[pl.ds(g * 16, 16)] = t + jnp.int32(1 << DROP_BITS)
            return carry

        lax.fori_loop(0, rpw // 16, group_body, 0)
        pltpu.sync_copy(lov, lo_hbm.at[wid])
        pltpu.sync_copy(hiv, hi_hbm.at[wid])

    lo, hi = k(x, y, z)
    return lo.reshape(B * N), hi.reshape(B * N)


def _pool_body(pr_ref, pt_ref, tp_ref, f3_ref, out_ref):
    q = pr_ref[0]
    pt = pt_ref[0]

    d2 = None
    for c in range(3):
        diff = q[:, c:c + 1] - pt[c:c + 1, :]
        sq = diff * diff
        d2 = sq if d2 is None else d2 + sq

    tlo = tp_ref[0][:, 0:1]
    thi = tp_ref[0][:, 1:2]
    ltf = jnp.where(d2 < tlo, 1.0, 0.0)
    lt2f = jnp.where(d2 < thi, 1.0, 0.0)
    eqf = lt2f - ltf
    m = jnp.sum(ltf, axis=1, keepdims=True)
    e = jnp.sum(eqf, axis=1, keepdims=True)
    tie_w = (float(K_NN) - m) / jnp.maximum(e, 1.0)
    w = ltf + eqf * tie_w

    out_ref[0] = jnp.dot(w, f3_ref[0], preferred_element_type=jnp.float32) \
        * (1.0 / K_NN)


def kernel(points, W1, b1, W2, b2, W3, b3):
    B, N, C = points.shape
    pr = jnp.pad(points, ((0, 0), (0, 0), (0, 8 - C)))
    pt = jnp.swapaxes(pr, 1, 2)
    w1p = jnp.pad(W1, ((0, 8 - C), (0, 0)))

    lo_bits, hi_bits = _sc_kth_bits(pt[:, 0, :], pt[:, 1, :], pt[:, 2, :])
    tlo = lax.bitcast_convert_type(lo_bits, jnp.float32).reshape(B, N, 1)
    thi = lax.bitcast_convert_type(hi_bits, jnp.float32).reshape(B, N, 1)
    tp = jnp.concatenate(
        [tlo, thi, jnp.zeros((B, N, 6), jnp.float32)], axis=-1)

    f3 = pl.pallas_call(
        _mlp_body,
        grid=(B,),
        in_specs=[
            pl.BlockSpec((1, N, 8), lambda b: (b, 0, 0)),
            pl.BlockSpec((8, 64), lambda b: (0, 0)),
            pl.BlockSpec((1, 64), lambda b: (0, 0)),
            pl.BlockSpec((64, 64), lambda b: (0, 0)),
            pl.BlockSpec((1, 64), lambda b: (0, 0)),
            pl.BlockSpec((64, 128), lambda b: (0, 0)),
            pl.BlockSpec((1, 128), lambda b: (0, 0)),
        ],
        out_specs=pl.BlockSpec((1, N, 128), lambda b: (b, 0, 0)),
        out_shape=jax.ShapeDtypeStruct((B, N, 128), jnp.float32),
    )(pr, w1p, b1[None], W2, b2[None], W3, b3[None])

    n_rb = N // ROW_BLOCK
    out = pl.pallas_call(
        _pool_body,
        grid=(B, n_rb),
        in_specs=[
            pl.BlockSpec((1, ROW_BLOCK, 8), lambda b, r: (b, r, 0)),
            pl.BlockSpec((1, 8, N), lambda b, r: (b, 0, 0)),
            pl.BlockSpec((1, ROW_BLOCK, 8), lambda b, r: (b, r, 0)),
            pl.BlockSpec((1, N, 128), lambda b, r: (b, 0, 0)),
        ],
        out_specs=pl.BlockSpec((1, ROW_BLOCK, 128), lambda b, r: (b, r, 0)),
        out_shape=jax.ShapeDtypeStruct((B, N, 128), jnp.float32),
    )(pr, pt, tp, f3)
    return out


# speedup vs baseline: 2.3496x; 2.3496x over previous
import functools

import jax
import jax.numpy as jnp
from jax.experimental import pallas as pl

K_NN = 32
ROW_BLOCK = 256
DROP_BITS = 14


def _mlp_body(pr_ref, w1_ref, b1_ref, w2_ref, b2_ref, w3_ref, b3_ref, f3_ref):
    p = pr_ref[0]
    f = jnp.maximum(jnp.dot(p, w1_ref[...], preferred_element_type=jnp.float32)
                    + b1_ref[...], 0.0)
    f = jnp.maximum(jnp.dot(f, w2_ref[...], preferred_element_type=jnp.float32)
                    + b2_ref[...], 0.0)
    f = jnp.maximum(jnp.dot(f, w3_ref[...], preferred_element_type=jnp.float32)
                    + b3_ref[...], 0.0)
    f3_ref[0] = f


def _knn_pool_body(pr_ref, pt_ref, f3_ref, out_ref):
    q = pr_ref[0]
    pt = pt_ref[0]

    d2 = None
    for c in range(3):
        diff = q[:, c:c + 1] - pt[c:c + 1, :]
        sq = diff * diff
        d2 = sq if d2 is None else d2 + sq

    bits = jax.lax.bitcast_convert_type(d2, jnp.int32)
    rb = bits.shape[0]
    n = bits.shape[1]
    ones = jnp.ones((n, 1), jnp.float32)

    hi16 = jnp.right_shift(bits, 16).astype(jnp.int16)

    def body16(i, carry):
        t, m = carry
        cand = t | jnp.left_shift(jnp.int32(1), 14 - i)
        x = (hi16 < cand.astype(jnp.int16)).astype(jnp.int16)
        while x.shape[1] > 128:
            h = x.shape[1] // 2
            x = x[:, :h] + x[:, h:]
        cnt = jnp.sum(x.astype(jnp.int32), axis=1, keepdims=True)
        take = cnt < K_NN
        return jnp.where(take, cand, t), jnp.where(take, cnt, m)

    t16, m16 = jax.lax.fori_loop(
        0, 15, body16,
        (jnp.zeros((rb, 1), jnp.int32), jnp.zeros((rb, 1), jnp.int32)))

    def body(i, carry):
        t, m = carry
        cand = t | jnp.left_shift(jnp.int32(1), 15 - i)
        cnt = jnp.sum((bits < cand).astype(jnp.int32), axis=1, keepdims=True)
        take = cnt < K_NN
        return jnp.where(take, cand, t), jnp.where(take, cnt, m)

    t0 = jnp.left_shift(t16, 16)
    t, m = jax.lax.fori_loop(0, 16 - DROP_BITS, body, (t0, m16))

    ltf = jnp.where(bits < t, 1.0, 0.0)
    lt2f = jnp.where(bits < t + jnp.int32(1 << DROP_BITS), 1.0, 0.0)
    eqf = lt2f - ltf
    e = jnp.sum(eqf, axis=1, keepdims=True)
    tie_w = (float(K_NN) - m.astype(jnp.float32)) / e
    w = ltf + eqf * tie_w

    out_ref[0] = jnp.dot(w, f3_ref[0], preferred_element_type=jnp.float32) \
        * (1.0 / K_NN)


def kernel(points, W1, b1, W2, b2, W3, b3):
    B, N, C = points.shape
    pr = jnp.pad(points, ((0, 0), (0, 0), (0, 8 - C)))
    pt = jnp.swapaxes(pr, 1, 2)
    w1p = jnp.pad(W1, ((0, 8 - C), (0, 0)))

    f3 = pl.pallas_call(
        _mlp_body,
        grid=(B,),
        in_specs=[
            pl.BlockSpec((1, N, 8), lambda b: (b, 0, 0)),
            pl.BlockSpec((8, 64), lambda b: (0, 0)),
            pl.BlockSpec((1, 64), lambda b: (0, 0)),
            pl.BlockSpec((64, 64), lambda b: (0, 0)),
            pl.BlockSpec((1, 64), lambda b: (0, 0)),
            pl.BlockSpec((64, 128), lambda b: (0, 0)),
            pl.BlockSpec((1, 128), lambda b: (0, 0)),
        ],
        out_specs=pl.BlockSpec((1, N, 128), lambda b: (b, 0, 0)),
        out_shape=jax.ShapeDtypeStruct((B, N, 128), jnp.float32),
    )(pr, w1p, b1[None], W2, b2[None], W3, b3[None])

    n_rb = N // ROW_BLOCK
    out = pl.pallas_call(
        _knn_pool_body,
        grid=(B, n_rb),
        in_specs=[
            pl.BlockSpec((1, ROW_BLOCK, 8), lambda b, r: (b, r, 0)),
            pl.BlockSpec((1, 8, N), lambda b, r: (b, 0, 0)),
            pl.BlockSpec((1, N, 128), lambda b, r: (b, 0, 0)),
        ],
        out_specs=pl.BlockSpec((1, ROW_BLOCK, 128), lambda b, r: (b, r, 0)),
        out_shape=jax.ShapeDtypeStruct((B, N, 128), jnp.float32),
    )(pr, pt, f3)
    return out
